# baseline (speedup 1.0000x reference)
import math

import jax
import jax.numpy as jnp
from jax.experimental import pallas as pl
from jax.experimental.pallas import tpu as pltpu

_B, _S, _D, _H, _TOPK = 1, 2048, 1024, 16, 64
_DH = _D // _H
_SCALE = 1.0 / math.sqrt(_DH)
_INT_MIN = -2147483648


def _fused_attn_kernel(x_ref, wqkv_ref, bqkv_ref, wout_ref, bout_ref, out_ref):
    h = pl.program_id(0)
    x = x_ref[...]
    w = wqkv_ref[0]
    b = bqkv_ref[0]
    qkv = jnp.dot(x, w.T, preferred_element_type=jnp.float32) + b
    q = qkv[:, :_DH]
    k = qkv[:, _DH:2 * _DH]
    v = qkv[:, 2 * _DH:]

    logits = jnp.dot(q, k.T, preferred_element_type=jnp.float32) * _SCALE

    ikey = jax.lax.bitcast_convert_type(logits, jnp.int32)
    key = jnp.where(ikey >= 0, ikey, ikey ^ jnp.int32(0x7FFFFFFF))

    gkey = jnp.max(key.reshape(key.shape[0], 16, 128), axis=1)
    gcnt0 = jnp.sum((gkey >= 0).astype(jnp.int32), axis=1, keepdims=True)
    gt = jnp.where(gcnt0 >= _TOPK, jnp.int32(0), jnp.int32(_INT_MIN))
    for bit in range(30, -1, -1):
        cand = gt | jnp.int32(1 << bit)
        gcnt = jnp.sum((gkey >= cand).astype(jnp.int32), axis=1, keepdims=True)
        gt = jnp.where(gcnt >= _TOPK, cand, gt)
    lo0 = gt
    hi0 = jnp.max(gkey, axis=1, keepdims=True) + 1

    def cond(carry):
        i, lo, hi = carry
        return jnp.logical_and(i < 40, jnp.max(hi - lo) > 1024)

    def body(carry):
        i, lo, hi = carry
        mid = lo + ((hi - lo) >> 1)
        cnt = jnp.sum((key >= mid).astype(jnp.int32), axis=1, keepdims=True)
        ge = cnt >= _TOPK
        exact = cnt == _TOPK
        lo = jnp.where(ge, mid, lo)
        hi = jnp.where(exact, mid + 1, jnp.where(ge, hi, mid))
        return i + 1, lo, hi

    _, t, _ = jax.lax.while_loop(cond, body, (jnp.int32(0), lo0, hi0))

    mkey = jnp.max(key, axis=1, keepdims=True)
    ikey2 = jnp.where(key >= 0, key, key ^ jnp.int32(0x7FFFFFFF))
    logits2 = jax.lax.bitcast_convert_type(ikey2, jnp.float32)
    m = jax.lax.bitcast_convert_type(
        jnp.where(mkey >= 0, mkey, mkey ^ jnp.int32(0x7FFFFFFF)), jnp.float32)
    wexp = jnp.where(key >= t, jnp.exp(logits2 - m), 0.0)
    denom = jnp.sum(wexp, axis=1, keepdims=True)
    wexp = wexp / denom
    attn = jnp.dot(wexp, v, preferred_element_type=jnp.float32)

    wo = wout_ref[0]
    contrib = jnp.dot(attn, wo, preferred_element_type=jnp.float32)

    @pl.when(h == 0)
    def _():
        out_ref[...] = x + bout_ref[...] + contrib

    @pl.when(h != 0)
    def _():
        out_ref[...] += contrib


def kernel(x, W_qkv, b_qkv, W_out, b_out):
    b, s, d = x.shape
    x2 = x.reshape(s, d)
    wqkv = W_qkv.reshape(_H, 3 * _DH, d)
    bqkv = b_qkv.reshape(_H, 1, 3 * _DH)
    wout_t = W_out.T.reshape(_H, _DH, d)
    bout = b_out.reshape(1, d)

    out = pl.pallas_call(
        _fused_attn_kernel,
        grid=(_H,),
        in_specs=[
            pl.BlockSpec((s, d), lambda h: (0, 0)),
            pl.BlockSpec((1, 3 * _DH, d), lambda h: (h, 0, 0)),
            pl.BlockSpec((1, 1, 3 * _DH), lambda h: (h, 0, 0)),
            pl.BlockSpec((1, _DH, d), lambda h: (h, 0, 0)),
            pl.BlockSpec((1, d), lambda h: (0, 0)),
        ],
        out_specs=pl.BlockSpec((s, d), lambda h: (0, 0)),
        out_shape=jax.ShapeDtypeStruct((s, d), jnp.float32),
        compiler_params=pltpu.CompilerParams(
            dimension_semantics=("arbitrary",),
            vmem_limit_bytes=110 * 1024 * 1024,
        ),
    )(x2, wqkv, bqkv, wout_t, bout)
    return out.reshape(b, s, d)


# speedup vs baseline: 1.3188x; 1.3188x over previous
import math

import jax
import jax.numpy as jnp
from jax.experimental import pallas as pl
from jax.experimental.pallas import tpu as pltpu

_B, _S, _D, _H, _TOPK = 1, 2048, 1024, 16, 64
_DH = _D // _H
_SCALE = 1.0 / math.sqrt(_DH)
_INT_MIN = -2147483648


def _fused_attn_kernel(x_ref, wqkv_ref, bqkv_ref, wout_ref, bout_ref, out_ref):
    h = pl.program_id(0)
    x = x_ref[...]
    w = wqkv_ref[0]
    b = bqkv_ref[0]
    qkv = jnp.dot(x, w.T, preferred_element_type=jnp.float32) + b
    q = qkv[:, :_DH]
    k = qkv[:, _DH:2 * _DH]
    v = qkv[:, 2 * _DH:]

    logits = jnp.dot(q, k.T, preferred_element_type=jnp.float32) * _SCALE

    ikey = jax.lax.bitcast_convert_type(logits, jnp.int32)
    key = jnp.where(ikey >= 0, ikey, ikey ^ jnp.int32(0x7FFFFFFF))

    gkey = jnp.max(key.reshape(key.shape[0], 16, 128), axis=1)
    gcnt0 = jnp.sum((gkey >= 0).astype(jnp.int32), axis=1, keepdims=True)
    gt = jnp.where(gcnt0 >= _TOPK, jnp.int32(0), jnp.int32(_INT_MIN))
    for bit in range(30, 15, -1):
        cand = gt | jnp.int32(1 << bit)
        gcnt = jnp.sum((gkey >= cand).astype(jnp.int32), axis=1, keepdims=True)
        gt = jnp.where(gcnt >= _TOPK, cand, gt)
    lo = gt
    hi = jnp.max(gkey, axis=1, keepdims=True) + 1

    for _ in range(15):
        mid = lo + ((hi - lo) >> 1)
        cnt = jnp.sum((key >= mid).astype(jnp.int32), axis=1, keepdims=True)
        ge = cnt >= _TOPK
        exact = cnt == _TOPK
        lo = jnp.where(ge, mid, lo)
        hi = jnp.where(exact, mid + 1, jnp.where(ge, hi, mid))
    t = lo

    mkey = jnp.max(key, axis=1, keepdims=True)
    ikey2 = jnp.where(key >= 0, key, key ^ jnp.int32(0x7FFFFFFF))
    logits2 = jax.lax.bitcast_convert_type(ikey2, jnp.float32)
    m = jax.lax.bitcast_convert_type(
        jnp.where(mkey >= 0, mkey, mkey ^ jnp.int32(0x7FFFFFFF)), jnp.float32)
    wexp = jnp.where(key >= t, jnp.exp(logits2 - m), 0.0)
    denom = jnp.sum(wexp, axis=1, keepdims=True)
    wexp = wexp / denom
    attn = jnp.dot(wexp, v, preferred_element_type=jnp.float32)

    wo = wout_ref[0]
    contrib = jnp.dot(attn, wo, preferred_element_type=jnp.float32)

    @pl.when(h == 0)
    def _():
        out_ref[...] = x + bout_ref[...] + contrib

    @pl.when(h != 0)
    def _():
        out_ref[...] += contrib


def kernel(x, W_qkv, b_qkv, W_out, b_out):
    b, s, d = x.shape
    x2 = x.reshape(s, d)
    wqkv = W_qkv.reshape(_H, 3 * _DH, d)
    bqkv = b_qkv.reshape(_H, 1, 3 * _DH)
    wout_t = W_out.T.reshape(_H, _DH, d)
    bout = b_out.reshape(1, d)

    out = pl.pallas_call(
        _fused_attn_kernel,
        grid=(_H,),
        in_specs=[
            pl.BlockSpec((s, d), lambda h: (0, 0)),
            pl.BlockSpec((1, 3 * _DH, d), lambda h: (h, 0, 0)),
            pl.BlockSpec((1, 1, 3 * _DH), lambda h: (h, 0, 0)),
            pl.BlockSpec((1, _DH, d), lambda h: (h, 0, 0)),
            pl.BlockSpec((1, d), lambda h: (0, 0)),
        ],
        out_specs=pl.BlockSpec((s, d), lambda h: (0, 0)),
        out_shape=jax.ShapeDtypeStruct((s, d), jnp.float32),
        compiler_params=pltpu.CompilerParams(
            dimension_semantics=("arbitrary",),
            vmem_limit_bytes=110 * 1024 * 1024,
        ),
    )(x2, wqkv, bqkv, wout_t, bout)
    return out.reshape(b, s, d)


# speedup vs baseline: 2.1903x; 1.6608x over previous
import math

import jax
import jax.numpy as jnp
from jax.experimental import pallas as pl
from jax.experimental.pallas import tpu as pltpu

_B, _S, _D, _H, _TOPK = 1, 2048, 1024, 16, 64
_DH = _D // _H
_SCALE = 1.0 / math.sqrt(_DH)
_INT_MIN = -2147483648


def _fused_attn_kernel(x_ref, wqkv_ref, bqkv_ref, wout_ref, bout_ref, out_ref):
    h = pl.program_id(0)
    x = x_ref[...]
    w = wqkv_ref[0]
    b = bqkv_ref[0]
    qkv = jnp.dot(x, w.T, preferred_element_type=jnp.float32) + b
    q = qkv[:, :_DH]
    k = qkv[:, _DH:2 * _DH]
    v = qkv[:, 2 * _DH:]

    logits_t = jax.lax.dot_general(
        k, q, dimension_numbers=(((1,), (1,)), ((), ())),
        preferred_element_type=jnp.float32) * _SCALE

    ikey = jax.lax.bitcast_convert_type(logits_t, jnp.int32)
    key = jnp.where(ikey >= 0, ikey, ikey ^ jnp.int32(0x7FFFFFFF))

    gkey = jnp.max(key.reshape(16, 128, key.shape[1]), axis=0)
    gcnt0 = jnp.sum((gkey >= 0).astype(jnp.int32), axis=0, keepdims=True)
    gt = jnp.where(gcnt0 >= _TOPK, jnp.int32(0), jnp.int32(_INT_MIN))
    for bit in range(30, 15, -1):
        cand = gt | jnp.int32(1 << bit)
        gcnt = jnp.sum((gkey >= cand).astype(jnp.int32), axis=0, keepdims=True)
        gt = jnp.where(gcnt >= _TOPK, cand, gt)
    lo = gt
    hi = jnp.max(gkey, axis=0, keepdims=True) + 1

    mhi = hi - 1
    for _ in range(15):
        mid = lo + ((hi - lo) >> 1)
        cnt = jnp.sum((key >= mid).astype(jnp.int32), axis=0, keepdims=True)
        ge = cnt >= _TOPK
        exact = cnt == _TOPK
        lo = jnp.where(ge, mid, lo)
        hi = jnp.where(exact, mid + 1, jnp.where(ge, hi, mid))
    t = lo

    ikey2 = jnp.where(key >= 0, key, key ^ jnp.int32(0x7FFFFFFF))
    logits2 = jax.lax.bitcast_convert_type(ikey2, jnp.float32)
    m = jax.lax.bitcast_convert_type(
        jnp.where(mhi >= 0, mhi, mhi ^ jnp.int32(0x7FFFFFFF)), jnp.float32)
    wexp = jnp.where(key >= t, jnp.exp(logits2 - m), 0.0)
    denom = jnp.sum(wexp, axis=0, keepdims=True)
    wexp = wexp * (1.0 / denom)
    attn = jax.lax.dot_general(
        wexp, v, dimension_numbers=(((0,), (0,)), ((), ())),
        preferred_element_type=jnp.float32)

    wo = wout_ref[0]
    contrib = jnp.dot(attn, wo, preferred_element_type=jnp.float32)

    @pl.when(h == 0)
    def _():
        out_ref[...] = x + bout_ref[...] + contrib

    @pl.when(h != 0)
    def _():
        out_ref[...] += contrib


def kernel(x, W_qkv, b_qkv, W_out, b_out):
    b, s, d = x.shape
    x2 = x.reshape(s, d)
    wqkv = W_qkv.reshape(_H, 3 * _DH, d)
    bqkv = b_qkv.reshape(_H, 1, 3 * _DH)
    wout_t = W_out.T.reshape(_H, _DH, d)
    bout = b_out.reshape(1, d)

    out = pl.pallas_call(
        _fused_attn_kernel,
        grid=(_H,),
        in_specs=[
            pl.BlockSpec((s, d), lambda h: (0, 0)),
            pl.BlockSpec((1, 3 * _DH, d), lambda h: (h, 0, 0)),
            pl.BlockSpec((1, 1, 3 * _DH), lambda h: (h, 0, 0)),
            pl.BlockSpec((1, _DH, d), lambda h: (h, 0, 0)),
            pl.BlockSpec((1, d), lambda h: (0, 0)),
        ],
        out_specs=pl.BlockSpec((s, d), lambda h: (0, 0)),
        out_shape=jax.ShapeDtypeStruct((s, d), jnp.float32),
        compiler_params=pltpu.CompilerParams(
            dimension_semantics=("arbitrary",),
            vmem_limit_bytes=110 * 1024 * 1024,
        ),
    )(x2, wqkv, bqkv, wout_t, bout)
    return out.reshape(b, s, d)


# speedup vs baseline: 2.4599x; 1.1231x over previous
import math

import jax
import jax.numpy as jnp
from jax.experimental import pallas as pl
from jax.experimental.pallas import tpu as pltpu

_B, _S, _D, _H, _TOPK = 1, 2048, 1024, 16, 64
_DH = _D // _H
_SCALE = 1.0 / math.sqrt(_DH)
_INT_MIN = -2147483648


def _fused_attn_kernel(x_ref, wqkv_ref, bqkv_ref, wout_ref, bout_ref, out_ref):
    h = pl.program_id(0)
    x = x_ref[...]
    w = wqkv_ref[0]
    b = bqkv_ref[0]
    qkv = jnp.dot(x, w.T, preferred_element_type=jnp.float32) + b
    q = qkv[:, :_DH]
    k = qkv[:, _DH:2 * _DH]
    v = qkv[:, 2 * _DH:]

    logits_t = jax.lax.dot_general(
        k, q, dimension_numbers=(((1,), (1,)), ((), ())),
        preferred_element_type=jnp.float32) * _SCALE

    def _key_to_f32(ck):
        ck = jnp.clip(ck, jnp.int32(-2139095040), jnp.int32(2139095040))
        ik = jnp.where(ck >= 0, ck, ck ^ jnp.int32(0x7FFFFFFF))
        return jax.lax.bitcast_convert_type(ik, jnp.float32)

    sq = logits_t.shape[1]
    gmaxf = jnp.max(logits_t.reshape(16, 128, sq), axis=0)
    gcnt0 = jnp.sum((gmaxf >= 0.0).astype(jnp.int32), axis=0, keepdims=True)
    gt = jnp.where(gcnt0 >= _TOPK, jnp.int32(0), jnp.int32(_INT_MIN))
    for bit in range(30, 15, -1):
        cand = gt | jnp.int32(1 << bit)
        gcnt = jnp.sum((gmaxf >= _key_to_f32(cand)).astype(jnp.int32),
                       axis=0, keepdims=True)
        gt = jnp.where(gcnt >= _TOPK, cand, gt)
    lo = gt

    m = jnp.max(gmaxf, axis=0, keepdims=True)
    im = jax.lax.bitcast_convert_type(m, jnp.int32)
    hi = jnp.where(im >= 0, im, im ^ jnp.int32(0x7FFFFFFF)) + 1

    for _ in range(13):
        mid = lo + ((hi - lo) >> 1)
        cnt = jnp.sum((logits_t >= _key_to_f32(mid)).astype(jnp.int32),
                      axis=0, keepdims=True)
        ge = cnt >= _TOPK
        exact = cnt == _TOPK
        lo = jnp.where(ge, mid, lo)
        hi = jnp.where(exact, mid + 1, jnp.where(ge, hi, mid))

    thr = _key_to_f32(lo)
    wexp = jnp.where(logits_t >= thr, jnp.exp(logits_t - m), 0.0)
    denom = jnp.sum(wexp, axis=0, keepdims=True)
    wexp = wexp * (1.0 / denom)
    attn = jax.lax.dot_general(
        wexp, v, dimension_numbers=(((0,), (0,)), ((), ())),
        preferred_element_type=jnp.float32)

    wo = wout_ref[0]
    contrib = jnp.dot(attn, wo, preferred_element_type=jnp.float32)

    @pl.when(h == 0)
    def _():
        out_ref[...] = x + bout_ref[...] + contrib

    @pl.when(h != 0)
    def _():
        out_ref[...] += contrib


def kernel(x, W_qkv, b_qkv, W_out, b_out):
    b, s, d = x.shape
    x2 = x.reshape(s, d)
    wqkv = W_qkv.reshape(_H, 3 * _DH, d)
    bqkv = b_qkv.reshape(_H, 1, 3 * _DH)
    wout_t = W_out.T.reshape(_H, _DH, d)
    bout = b_out.reshape(1, d)

    out = pl.pallas_call(
        _fused_attn_kernel,
        grid=(_H,),
        in_specs=[
            pl.BlockSpec((s, d), lambda h: (0, 0)),
            pl.BlockSpec((1, 3 * _DH, d), lambda h: (h, 0, 0)),
            pl.BlockSpec((1, 1, 3 * _DH), lambda h: (h, 0, 0)),
            pl.BlockSpec((1, _DH, d), lambda h: (h, 0, 0)),
            pl.BlockSpec((1, d), lambda h: (0, 0)),
        ],
        out_specs=pl.BlockSpec((s, d), lambda h: (0, 0)),
        out_shape=jax.ShapeDtypeStruct((s, d), jnp.float32),
        compiler_params=pltpu.CompilerParams(
            dimension_semantics=("arbitrary",),
            vmem_limit_bytes=110 * 1024 * 1024,
        ),
    )(x2, wqkv, bqkv, wout_t, bout)
    return out.reshape(b, s, d)


# speedup vs baseline: 3.7041x; 1.5057x over previous
import math

import jax
import jax.numpy as jnp
from jax.experimental import pallas as pl
from jax.experimental.pallas import tpu as pltpu

_B, _S, _D, _H, _TOPK = 1, 2048, 1024, 16, 64
_DH = _D // _H
_SCALE = 1.0 / math.sqrt(_DH)
_INT_MIN = -2147483648


def _fused_attn_kernel(x_ref, wqkv_ref, bqkv_ref, wout_ref, bout_ref, out_ref):
    h = pl.program_id(0)
    x = x_ref[...]
    w = wqkv_ref[0]
    b = bqkv_ref[0]
    qkv = jnp.dot(x, w.T, preferred_element_type=jnp.float32) + b
    q = qkv[:, :_DH]
    k = qkv[:, _DH:2 * _DH]
    v = qkv[:, 2 * _DH:]

    logits_t = jax.lax.dot_general(
        k, q, dimension_numbers=(((1,), (1,)), ((), ())),
        preferred_element_type=jnp.float32) * _SCALE

    def _key_to_f32(ck):
        ck = jnp.clip(ck, jnp.int32(-2139095040), jnp.int32(2139095040))
        ik = jnp.where(ck >= 0, ck, ck ^ jnp.int32(0x7FFFFFFF))
        return jax.lax.bitcast_convert_type(ik, jnp.float32)

    sq = logits_t.shape[1]
    tile_bf = logits_t.astype(jnp.bfloat16)
    gmaxb = jnp.max(tile_bf.reshape(16, 128, sq), axis=0)
    gcnt0 = jnp.sum(
        jnp.where(gmaxb >= jnp.bfloat16(0.0), jnp.bfloat16(1), jnp.bfloat16(0)),
        axis=0, keepdims=True, dtype=jnp.bfloat16).astype(jnp.int32)
    gt = jnp.where(gcnt0 >= _TOPK, jnp.int32(0), jnp.int32(_INT_MIN))
    for bit in range(30, 15, -1):
        cand = gt | jnp.int32(1 << bit)
        cb = _key_to_f32(cand).astype(jnp.bfloat16)
        gcnt = jnp.sum(
            jnp.where(gmaxb >= cb, jnp.bfloat16(1), jnp.bfloat16(0)),
            axis=0, keepdims=True, dtype=jnp.bfloat16).astype(jnp.int32)
        gt = jnp.where(gcnt >= _TOPK, cand, gt)
    lo = jnp.maximum(gt, jnp.int32(_INT_MIN + 32768)) - 32768

    m = jnp.max(gmaxb.astype(jnp.float32), axis=0, keepdims=True)
    im = jax.lax.bitcast_convert_type(m, jnp.int32)
    hi = jnp.where(im >= 0, im, im ^ jnp.int32(0x7FFFFFFF)) + 1

    for _ in range(6):
        mid = lo + ((hi - lo) >> 1)
        midb = _key_to_f32(mid).astype(jnp.bfloat16)
        sel = jnp.where(tile_bf >= midb, jnp.bfloat16(1), jnp.bfloat16(0))
        part = jnp.sum(sel.reshape(16, 128, sq), axis=1,
                       dtype=jnp.bfloat16)
        cnt = jnp.sum(part.astype(jnp.float32), axis=0,
                      keepdims=True).astype(jnp.int32)
        ge = cnt >= _TOPK
        lo = jnp.where(ge, jnp.maximum(mid, jnp.int32(_INT_MIN + 32768)) - 32768,
                       lo)
        hi = jnp.where(ge, hi, mid)

    for _ in range(6):
        mid = lo + ((hi - lo) >> 1)
        cnt = jnp.sum((logits_t >= _key_to_f32(mid)).astype(jnp.int32),
                      axis=0, keepdims=True)
        ge = cnt >= _TOPK
        exact = cnt == _TOPK
        lo = jnp.where(ge, mid, lo)
        hi = jnp.where(exact, mid + 1, jnp.where(ge, hi, mid))

    thr = _key_to_f32(lo)
    wexp = jnp.where(logits_t >= thr, jnp.exp(logits_t - m), 0.0)

    col = jax.lax.broadcasted_iota(jnp.int32, (v.shape[0], 2 * _DH), 1)
    v_aug = jax.lax.pad(v, jnp.float32(0.0), ((0, 0, 0), (0, _DH, 0)))
    v_aug = v_aug + (col == _DH).astype(jnp.float32)
    attn_aug = jax.lax.dot_general(
        wexp, v_aug, dimension_numbers=(((0,), (0,)), ((), ())),
        preferred_element_type=jnp.float32)
    attn = attn_aug[:, :_DH] / attn_aug[:, _DH:_DH + 1]

    wo = wout_ref[0]
    contrib = jnp.dot(attn, wo, preferred_element_type=jnp.float32)

    @pl.when(h == 0)
    def _():
        out_ref[...] = x + bout_ref[...] + contrib

    @pl.when(h != 0)
    def _():
        out_ref[...] += contrib


def kernel(x, W_qkv, b_qkv, W_out, b_out):
    b, s, d = x.shape
    x2 = x.reshape(s, d)
    wqkv = W_qkv.reshape(_H, 3 * _DH, d)
    bqkv = b_qkv.reshape(_H, 1, 3 * _DH)
    wout_t = W_out.T.reshape(_H, _DH, d)
    bout = b_out.reshape(1, d)

    out = pl.pallas_call(
        _fused_attn_kernel,
        grid=(_H,),
        in_specs=[
            pl.BlockSpec((s, d), lambda h: (0, 0)),
            pl.BlockSpec((1, 3 * _DH, d), lambda h: (h, 0, 0)),
            pl.BlockSpec((1, 1, 3 * _DH), lambda h: (h, 0, 0)),
            pl.BlockSpec((1, _DH, d), lambda h: (h, 0, 0)),
            pl.BlockSpec((1, d), lambda h: (0, 0)),
        ],
        out_specs=pl.BlockSpec((s, d), lambda h: (0, 0)),
        out_shape=jax.ShapeDtypeStruct((s, d), jnp.float32),
        compiler_params=pltpu.CompilerParams(
            dimension_semantics=("arbitrary",),
            vmem_limit_bytes=110 * 1024 * 1024,
        ),
    )(x2, wqkv, bqkv, wout_t, bout)
    return out.reshape(b, s, d)
